# Initial kernel scaffold; baseline (speedup 1.0000x reference)
import jax
import jax.numpy as jnp
from jax.experimental import pallas as pl


def kernel(t, gamma):
    raise NotImplementedError("write your pallas kernel here")


# speedup vs baseline: 4.4808x; 4.4808x over previous
import functools

import jax
import jax.numpy as jnp
from jax import lax
from jax.experimental import pallas as pl
from jax.experimental.pallas import tpu as pltpu
from jax.experimental.pallas import tpu_sc as plsc

_INFO = plsc.get_sparse_core_info()
_NC = _INFO.num_cores
_NS = _INFO.num_subcores
_L = _INFO.num_lanes
_NW = _NC * _NS


@functools.lru_cache(maxsize=None)
def _make(B: int, G: int):
    b_per_w = B // _NW
    assert b_per_w * _NW == B and b_per_w % _L == 0
    g_pad = (G + 15) // 16 * 16
    mesh = plsc.VectorSubcoreMesh(core_axis_name="c", subcore_axis_name="s")

    @functools.partial(
        pl.kernel,
        mesh=mesh,
        out_type=jax.ShapeDtypeStruct((B,), jnp.float32),
        scratch_types=[
            pltpu.VMEM((g_pad,), jnp.float32),
            pltpu.VMEM((b_per_w,), jnp.float32),
            pltpu.VMEM((b_per_w,), jnp.float32),
        ],
        compiler_params=pltpu.CompilerParams(needs_layout_passes=False),
    )
    def k(t_hbm, gamma_hbm, out_hbm, gamma_v, t_v, out_v):
        wid = lax.axis_index("s") * _NC + lax.axis_index("c")
        base = wid * b_per_w
        pltpu.sync_copy(gamma_hbm, gamma_v.at[pl.ds(0, G)])
        pltpu.sync_copy(t_hbm.at[pl.ds(base, b_per_w)], t_v)
        scale = jnp.float32(G - 1)
        hi = jnp.int32(G - 1)
        lo = jnp.int32(0)
        for i in range(b_per_w // _L):
            tv = t_v[pl.ds(i * _L, _L)]
            idx = (tv * scale).astype(jnp.int32)
            idx = jnp.minimum(jnp.maximum(idx, lo), hi)
            out_v[pl.ds(i * _L, _L)] = plsc.load_gather(gamma_v, [idx])
        pltpu.sync_copy(out_v, out_hbm.at[pl.ds(base, b_per_w)])

    return k


@jax.jit
def kernel(t, gamma):
    return _make(t.shape[0], gamma.shape[0])(t, gamma)


# speedup vs baseline: 4.5833x; 1.0229x over previous
import functools

import jax
import jax.numpy as jnp
from jax import lax
from jax.experimental import pallas as pl
from jax.experimental.pallas import tpu as pltpu
from jax.experimental.pallas import tpu_sc as plsc

_INFO = plsc.get_sparse_core_info()
_NC = _INFO.num_cores
_NS = _INFO.num_subcores
_L = _INFO.num_lanes
_NW = _NC * _NS


@functools.lru_cache(maxsize=None)
def _make(B: int, G: int):
    b_per_w = B // _NW
    assert b_per_w * _NW == B and b_per_w % _L == 0
    g_pad = (G + 15) // 16 * 16
    mesh = plsc.VectorSubcoreMesh(core_axis_name="c", subcore_axis_name="s")

    @functools.partial(
        pl.kernel,
        mesh=mesh,
        out_type=jax.ShapeDtypeStruct((B,), jnp.float32),
        scratch_types=[
            pltpu.VMEM((g_pad,), jnp.float32),
            pltpu.VMEM((b_per_w,), jnp.float32),
            pltpu.VMEM((b_per_w,), jnp.float32),
            pltpu.SemaphoreType.DMA,
            pltpu.SemaphoreType.DMA,
        ],
        compiler_params=pltpu.CompilerParams(needs_layout_passes=False),
    )
    def k(t_hbm, gamma_hbm, out_hbm, gamma_v, t_v, out_v, sem_g, sem_t):
        wid = lax.axis_index("s") * _NC + lax.axis_index("c")
        base = wid * b_per_w
        cp_g = pltpu.async_copy(gamma_hbm, gamma_v.at[pl.ds(0, G)], sem_g)
        cp_t = pltpu.async_copy(t_hbm.at[pl.ds(base, b_per_w)], t_v, sem_t)
        cp_g.wait()
        cp_t.wait()
        scale = jnp.float32(G - 1)
        hi = jnp.int32(G - 1)
        lo = jnp.int32(0)
        for i in range(b_per_w // _L):
            tv = t_v[pl.ds(i * _L, _L)]
            idx = (tv * scale).astype(jnp.int32)
            idx = jnp.minimum(jnp.maximum(idx, lo), hi)
            out_v[pl.ds(i * _L, _L)] = plsc.load_gather(gamma_v, [idx])
        pltpu.sync_copy(out_v, out_hbm.at[pl.ds(base, b_per_w)])

    return k


@jax.jit
def kernel(t, gamma):
    return _make(t.shape[0], gamma.shape[0])(t, gamma)


# speedup vs baseline: 4.6307x; 1.0103x over previous
import functools

import jax
import jax.numpy as jnp
from jax import lax
from jax.experimental import pallas as pl
from jax.experimental.pallas import tpu as pltpu
from jax.experimental.pallas import tpu_sc as plsc

_INFO = plsc.get_sparse_core_info()
_NC = _INFO.num_cores
_NS = _INFO.num_subcores
_L = _INFO.num_lanes
_NW = _NC * _NS


@functools.lru_cache(maxsize=None)
def _make(B: int, G: int):
    b_per_w = B // _NW
    assert b_per_w * _NW == B and b_per_w % _L == 0
    g_pad = (G + 15) // 16 * 16
    mesh = plsc.VectorSubcoreMesh(core_axis_name="c", subcore_axis_name="s")

    @functools.partial(
        pl.kernel,
        mesh=mesh,
        out_type=jax.ShapeDtypeStruct((B,), jnp.float32),
        scratch_types=[
            pltpu.VMEM((g_pad,), jnp.float32),
            pltpu.VMEM((b_per_w,), jnp.float32),
            pltpu.VMEM((b_per_w,), jnp.float32),
            pltpu.SemaphoreType.DMA,
            pltpu.SemaphoreType.DMA,
        ],
        compiler_params=pltpu.CompilerParams(needs_layout_passes=False),
    )
    def k(t_hbm, gamma_hbm, out_hbm, gamma_v, t_v, out_v, sem_g, sem_t):
        wid = lax.axis_index("s") * _NC + lax.axis_index("c")
        base = wid * b_per_w
        cp_g = pltpu.async_copy(gamma_hbm, gamma_v.at[pl.ds(0, G)], sem_g)
        cp_t = pltpu.async_copy(t_hbm.at[pl.ds(base, b_per_w)], t_v, sem_t)
        scale = jnp.float32(G - 1)
        hi = jnp.int32(G - 1)
        lo = jnp.int32(0)
        n_chunks = b_per_w // _L
        cp_t.wait()
        idxs = []
        for i in range(n_chunks):
            tv = t_v[pl.ds(i * _L, _L)]
            idx = (tv * scale).astype(jnp.int32)
            idxs.append(jnp.minimum(jnp.maximum(idx, lo), hi))
        cp_g.wait()
        half = n_chunks // 2
        for i in range(half):
            out_v[pl.ds(i * _L, _L)] = plsc.load_gather(gamma_v, [idxs[i]])
        cp_o0 = pltpu.async_copy(
            out_v.at[pl.ds(0, half * _L)],
            out_hbm.at[pl.ds(base, half * _L)], sem_t)
        for i in range(half, n_chunks):
            out_v[pl.ds(i * _L, _L)] = plsc.load_gather(gamma_v, [idxs[i]])
        cp_o1 = pltpu.async_copy(
            out_v.at[pl.ds(half * _L, half * _L)],
            out_hbm.at[pl.ds(base + half * _L, half * _L)], sem_g)
        cp_o0.wait()
        cp_o1.wait()

    return k


@jax.jit
def kernel(t, gamma):
    return _make(t.shape[0], gamma.shape[0])(t, gamma)


# speedup vs baseline: 4.7355x; 1.0226x over previous
import functools

import jax
import jax.numpy as jnp
from jax import lax
from jax.experimental import pallas as pl
from jax.experimental.pallas import tpu as pltpu
from jax.experimental.pallas import tpu_sc as plsc

_INFO = plsc.get_sparse_core_info()
_NC = _INFO.num_cores
_NS = _INFO.num_subcores
_L = _INFO.num_lanes
_NW = _NC * _NS


@functools.lru_cache(maxsize=None)
def _make(B: int, G: int):
    b_per_w = B // _NW
    assert b_per_w * _NW == B and b_per_w % _L == 0
    g_pad = (G + 15) // 16 * 16
    mesh = plsc.VectorSubcoreMesh(core_axis_name="c", subcore_axis_name="s")

    @functools.partial(
        pl.kernel,
        mesh=mesh,
        out_type=jax.ShapeDtypeStruct((B,), jnp.float32),
        scratch_types=[
            pltpu.VMEM((g_pad,), jnp.float32),
            pltpu.VMEM((b_per_w,), jnp.float32),
            pltpu.VMEM((b_per_w,), jnp.float32),
            pltpu.SemaphoreType.DMA,
            pltpu.SemaphoreType.DMA,
        ],
        compiler_params=pltpu.CompilerParams(needs_layout_passes=False),
    )
    def k(t_hbm, gamma_hbm, out_hbm, gamma_v, t_v, out_v, sem_g, sem_t):
        wid = lax.axis_index("s") * _NC + lax.axis_index("c")
        base = wid * b_per_w
        cp_g = pltpu.async_copy(gamma_hbm, gamma_v.at[pl.ds(0, G)], sem_g)
        cp_t = pltpu.async_copy(t_hbm.at[pl.ds(base, b_per_w)], t_v, sem_t)
        scale = jnp.float32(G - 1)
        hi = jnp.int32(G - 1)
        lo = jnp.int32(0)
        n_chunks = b_per_w // _L
        half = n_chunks // 2
        cp_t.wait()
        cp_g.wait()

        def body(i):
            tv = t_v[pl.ds(i * _L, _L)]
            idx = (tv * scale).astype(jnp.int32)
            idx = jnp.minimum(jnp.maximum(idx, lo), hi)
            out_v[pl.ds(i * _L, _L)] = plsc.load_gather(gamma_v, [idx])

        plsc.parallel_loop(0, half, unroll=8)(body)
        cp_o0 = pltpu.async_copy(
            out_v.at[pl.ds(0, half * _L)],
            out_hbm.at[pl.ds(base, half * _L)], sem_t)
        plsc.parallel_loop(half, n_chunks, unroll=8)(body)
        cp_o1 = pltpu.async_copy(
            out_v.at[pl.ds(half * _L, half * _L)],
            out_hbm.at[pl.ds(base + half * _L, half * _L)], sem_g)
        cp_o0.wait()
        cp_o1.wait()

    return k


@jax.jit
def kernel(t, gamma):
    return _make(t.shape[0], gamma.shape[0])(t, gamma)
